# baseline (speedup 1.0000x reference)
import functools

import jax
import jax.numpy as jnp
import numpy as np
from jax import lax
from jax.experimental import pallas as pl
from jax.experimental.pallas import tpu as pltpu
from jax.experimental.pallas import tpu_sc as plsc

D = 128
H = D // 2
L = 32

_NC = 2
_NS = 16
_NW = _NC * _NS

_CH = 512
_NSLICE = 4
_HIMASK = np.int32(-65536)


def _pack_bf16_pair(lo_f32, hi_f32):
    lo_b = lax.bitcast_convert_type(
        lo_f32.astype(jnp.bfloat16).astype(jnp.float32), jnp.int32)
    hi_b = lax.bitcast_convert_type(
        hi_f32.astype(jnp.bfloat16).astype(jnp.float32), jnp.int32)
    return lax.bitwise_or(lax.shift_right_logical(lo_b, 16),
                          lax.bitwise_and(hi_b, _HIMASK))


def _unpack_even_odd(packed_i32):
    even = lax.bitcast_convert_type(
        lax.shift_left(packed_i32, 16), jnp.float32)
    odd = lax.bitcast_convert_type(
        lax.bitwise_and(packed_i32, _HIMASK), jnp.float32)
    return even, odd


def _proj_kernel(feat_ref, a_ref, w1a_ref, p_ref, f1_ref):
    f = feat_ref[...]
    p32 = jnp.dot(f, a_ref[...], preferred_element_type=jnp.float32)
    p_ref[...] = _pack_bf16_pair(p32[:, :H], p32[:, H:])
    f1_ref[...] = jnp.dot(f, w1a_ref[...], preferred_element_type=jnp.float32)


def _ctab_kernel(r_ref, bm_ref, bgv_ref, c_ref):
    c_ref[...] = (
        jnp.dot(r_ref[...], bm_ref[...], preferred_element_type=jnp.float32)
        + bgv_ref[...]
    )


def _combine_kernel(g_ref, ra_ref, rb_ref, ce_ref, co_ref, s1_ref,
                    w1be_ref, w1bo_ref, b1_ref, out_ref):
    g2 = g_ref[...]
    e_a, o_a = _unpack_even_odd(g2[:, :H])
    e_b, o_b = _unpack_even_odd(g2[:, H:])
    npair = e_a.shape[0]
    ra = ra_ref[0, 0, :]
    rb = rb_ref[0, 0, :]
    iota8 = lax.broadcasted_iota(jnp.int32, (npair, 8), 1)
    oha = (ra[:, None] == iota8).astype(jnp.float32)
    ohb = (rb[:, None] == iota8).astype(jnp.float32)
    ce = ce_ref[...]
    co = co_ref[...]
    h_ae = jnp.maximum(
        e_a + jnp.dot(oha, ce, preferred_element_type=jnp.float32), 0.0)
    h_ao = jnp.maximum(
        o_a + jnp.dot(oha, co, preferred_element_type=jnp.float32), 0.0)
    h_be = jnp.maximum(
        e_b + jnp.dot(ohb, ce, preferred_element_type=jnp.float32), 0.0)
    h_bo = jnp.maximum(
        o_b + jnp.dot(ohb, co, preferred_element_type=jnp.float32), 0.0)
    n_e = jnp.sum((h_ae + h_be).reshape(-1, L // 2, H), axis=1) * (1.0 / L)
    n_o = jnp.sum((h_ao + h_bo).reshape(-1, L // 2, H), axis=1) * (1.0 / L)
    comb = (s1_ref[...]
            + jnp.dot(n_e, w1be_ref[...], preferred_element_type=jnp.float32)
            + jnp.dot(n_o, w1bo_ref[...], preferred_element_type=jnp.float32)
            + b1_ref[...])
    out_ref[...] = jnp.maximum(comb, 0.0)


def _sc_gather_body(p_hbm, f1_hbm, uv_hbm, nodes_hbm, g_out, s_out,
                    uvidx_v, buf_a, buf_b, rows_s, nidx_v, sem):
    wid = lax.axis_index("s") * _NC + lax.axis_index("c")
    bpw = uv_hbm.shape[1]
    nch = bpw * 128 // _CH
    spw = nodes_hbm.shape[0] * 128 // _NW
    ipc = _CH // 128

    pltpu.sync_copy(uv_hbm.at[wid], uvidx_v)

    def fire(buf, t):
        for j in range(ipc):
            pltpu.async_copy(p_hbm.at[uvidx_v.at[t * ipc + j]],
                             buf.at[pl.ds(j * 128, 128)], sem)

    def drain(buf):
        pltpu.make_async_copy(p_hbm.at[pl.ds(0, _CH)], buf, sem).wait()

    def writeback(buf, t):
        pltpu.sync_copy(buf, g_out.at[pl.ds(wid * bpw * 128 + t * _CH, _CH)])

    fire(buf_a, 0)

    def two_chunks(i, carry):
        t0 = i * 2
        fire(buf_b, t0 + 1)
        drain(buf_a)
        writeback(buf_a, t0)

        @pl.when(i < nch // 2 - 1)
        def _():
            fire(buf_a, t0 + 2)

        drain(buf_b)
        writeback(buf_b, t0 + 1)
        return carry

    lax.fori_loop(0, nch // 2, two_chunks, 0)

    pltpu.sync_copy(nodes_hbm.at[pl.ds(wid * (spw // 128), spw // 128)],
                    nidx_v)
    for h in range(spw // 128):
        pltpu.async_copy(f1_hbm.at[nidx_v.at[h]], rows_s, sem)
        pltpu.make_async_copy(f1_hbm.at[pl.ds(0, 128)], rows_s, sem).wait()
        pltpu.sync_copy(rows_s, s_out.at[pl.ds(wid * spw + h * 128, 128)])


def kernel(nodes, history_uv, history_r, feat_table, r_table, W_gv, b_gv, W1, b1):
    B = nodes.shape[0]
    V = feat_table.shape[0]
    BS = B // _NSLICE
    BLS = BS * L
    bpw = BLS // _NW // 128

    nodes_i = nodes.astype(jnp.int32).reshape(_NSLICE, BS // 128, 128)
    uv_i = history_uv.astype(jnp.int32).reshape(_NSLICE, _NW, bpw, 128)
    hr = history_r.astype(jnp.int32)
    npair = 128 * L // 2
    ra_i = hr[:, 0::2].reshape(_NSLICE, BS // 128, 1, npair)
    rb_i = hr[:, 1::2].reshape(_NSLICE, BS // 128, 1, npair)

    A = W_gv[:D]
    A_perm = jnp.concatenate([A[:, 0::2], A[:, 1::2]], axis=1)
    Bm = W_gv[D:]
    W1a = W1[:D]
    W1b = W1[D:]
    W1be = W1b[0::2, :]
    W1bo = W1b[1::2, :]
    r_pad = jnp.pad(r_table, ((0, 8 - r_table.shape[0]), (0, 0)))

    rb = 10000
    P, F1 = pl.pallas_call(
        _proj_kernel,
        grid=(V // rb,),
        in_specs=[
            pl.BlockSpec((rb, D), lambda i: (i, 0)),
            pl.BlockSpec((D, D), lambda i: (0, 0)),
            pl.BlockSpec((D, D), lambda i: (0, 0)),
        ],
        out_specs=[
            pl.BlockSpec((rb, H), lambda i: (i, 0)),
            pl.BlockSpec((rb, D), lambda i: (i, 0)),
        ],
        out_shape=[
            jax.ShapeDtypeStruct((V, H), jnp.int32),
            jax.ShapeDtypeStruct((V, D), jnp.float32),
        ],
    )(feat_table, A_perm, W1a)

    C = pl.pallas_call(
        _ctab_kernel,
        out_shape=jax.ShapeDtypeStruct((8, D), jnp.float32),
    )(r_pad, Bm, b_gv.reshape(1, D))
    Ce = C[:, 0::2]
    Co = C[:, 1::2]

    mesh = plsc.VectorSubcoreMesh(core_axis_name="c", subcore_axis_name="s")
    sc_gather = functools.partial(
        pl.kernel,
        mesh=mesh,
        compiler_params=pltpu.CompilerParams(use_tc_tiling_on_sc=False),
        out_type=(
            jax.ShapeDtypeStruct((BLS, H), jnp.int32),
            jax.ShapeDtypeStruct((BS, D), jnp.float32),
        ),
        scratch_types=[
            pltpu.VMEM((bpw, 128), jnp.int32),
            pltpu.VMEM((_CH, H), jnp.int32),
            pltpu.VMEM((_CH, H), jnp.int32),
            pltpu.VMEM((128, D), jnp.float32),
            pltpu.VMEM((max(BS // _NW // 128, 1), 128), jnp.int32),
            pltpu.SemaphoreType.DMA,
        ],
    )(_sc_gather_body)

    rblp = 128 * L // 2
    b1r = b1.reshape(1, D)
    outs = []
    for s in range(_NSLICE):
        G, S1 = sc_gather(P, F1, uv_i[s], nodes_i[s])
        G2 = G.reshape(BLS // 2, D)
        outs.append(pl.pallas_call(
            _combine_kernel,
            grid=(BS // 128,),
            in_specs=[
                pl.BlockSpec((rblp, D), lambda i: (i, 0)),
                pl.BlockSpec((1, 1, rblp), lambda i: (i, 0, 0)),
                pl.BlockSpec((1, 1, rblp), lambda i: (i, 0, 0)),
                pl.BlockSpec((8, H), lambda i: (0, 0)),
                pl.BlockSpec((8, H), lambda i: (0, 0)),
                pl.BlockSpec((128, D), lambda i: (i, 0)),
                pl.BlockSpec((H, D), lambda i: (0, 0)),
                pl.BlockSpec((H, D), lambda i: (0, 0)),
                pl.BlockSpec((1, D), lambda i: (0, 0)),
            ],
            out_specs=pl.BlockSpec((128, D), lambda i: (i, 0)),
            out_shape=jax.ShapeDtypeStruct((BS, D), jnp.float32),
        )(G2, ra_i[s], rb_i[s], Ce, Co, S1, W1be, W1bo, b1r))
    return jnp.concatenate(outs, axis=0)


# speedup vs baseline: 1.1754x; 1.1754x over previous
import functools

import jax
import jax.numpy as jnp
from jax import lax
from jax.experimental import pallas as pl
from jax.experimental.pallas import tpu as pltpu
from jax.experimental.pallas import tpu_sc as plsc

D = 128
L = 32

_NC = 2
_NS = 16
_NW = _NC * _NS

_CH = 256


def _proj_kernel(feat_ref, a_ref, w1a_ref, p_ref, f1_ref):
    f = feat_ref[...]
    p_ref[...] = jnp.dot(f, a_ref[...], preferred_element_type=jnp.float32)
    f1_ref[...] = jnp.dot(f, w1a_ref[...], preferred_element_type=jnp.float32)


def _ctab_kernel(r_ref, bm_ref, bgv_ref, c_ref):
    c_ref[...] = (
        jnp.dot(r_ref[...], bm_ref[...], preferred_element_type=jnp.float32)
        + bgv_ref[...]
    )


def _combine_kernel(g_ref, r_ref, c_ref, s1_ref, w1b_ref, b1_ref, out_ref):
    g = g_ref[...]
    r = r_ref[0, 0, :]
    oh = (r[:, None] == lax.broadcasted_iota(jnp.int32, (r.shape[0], 8), 1))
    rc = jnp.dot(oh.astype(jnp.float32), c_ref[...],
                 preferred_element_type=jnp.float32)
    h = jnp.maximum(g + rc, 0.0)
    neigh = jnp.sum(h.reshape(-1, L, D), axis=1) * (1.0 / L)
    comb = (s1_ref[...]
            + jnp.dot(neigh, w1b_ref[...], preferred_element_type=jnp.float32)
            + b1_ref[...])
    out_ref[...] = jnp.maximum(comb, 0.0)


def _sc_gather_body(p_hbm, f1_hbm, uv_hbm, nodes_hbm, g_out, s_out,
                    uvidx_v, buf_a, buf_b, rows_s, nidx_v, sem):
    wid = lax.axis_index("s") * _NC + lax.axis_index("c")
    bpw = uv_hbm.shape[1]
    nch = bpw * 128 // _CH
    spw = nodes_hbm.shape[0] * 128 // _NW
    ipc = _CH // 128

    pltpu.sync_copy(uv_hbm.at[wid], uvidx_v)

    def fire(buf, t):
        for j in range(ipc):
            pltpu.async_copy(p_hbm.at[uvidx_v.at[t * ipc + j]],
                             buf.at[pl.ds(j * 128, 128)], sem)

    def drain(buf):
        pltpu.make_async_copy(p_hbm.at[pl.ds(0, _CH)], buf, sem).wait()

    def writeback(buf, t):
        pltpu.sync_copy(buf, g_out.at[pl.ds(wid * bpw * 128 + t * _CH, _CH)])

    fire(buf_a, 0)

    def two_chunks(i, carry):
        t0 = i * 2
        fire(buf_b, t0 + 1)
        drain(buf_a)
        writeback(buf_a, t0)

        @pl.when(i < nch // 2 - 1)
        def _():
            fire(buf_a, t0 + 2)

        drain(buf_b)
        writeback(buf_b, t0 + 1)
        return carry

    lax.fori_loop(0, nch // 2, two_chunks, 0)

    pltpu.sync_copy(nodes_hbm.at[pl.ds(wid * (spw // 128), spw // 128)],
                    nidx_v)
    for h in range(spw // 128):
        pltpu.async_copy(f1_hbm.at[nidx_v.at[h]], rows_s, sem)
        pltpu.make_async_copy(f1_hbm.at[pl.ds(0, 128)], rows_s, sem).wait()
        pltpu.sync_copy(rows_s, s_out.at[pl.ds(wid * spw + h * 128, 128)])


_NSLICE = 4


def kernel(nodes, history_uv, history_r, feat_table, r_table, W_gv, b_gv, W1, b1):
    B = nodes.shape[0]
    V = feat_table.shape[0]
    BS = B // _NSLICE
    BLS = BS * L
    bpw = BLS // _NW // 128

    nodes_i = nodes.astype(jnp.int32).reshape(_NSLICE, BS // 128, 128)
    uv_i = history_uv.astype(jnp.int32).reshape(_NSLICE, _NW, bpw, 128)
    r3 = history_r.astype(jnp.int32).reshape(_NSLICE, BS // 128, 1, 128 * L)

    A = W_gv[:D]
    Bm = W_gv[D:]
    W1a = W1[:D]
    W1b = W1[D:]
    r_pad = jnp.pad(r_table, ((0, 8 - r_table.shape[0]), (0, 0)))

    rb = 10000
    P, F1 = pl.pallas_call(
        _proj_kernel,
        grid=(V // rb,),
        in_specs=[
            pl.BlockSpec((rb, D), lambda i: (i, 0)),
            pl.BlockSpec((D, D), lambda i: (0, 0)),
            pl.BlockSpec((D, D), lambda i: (0, 0)),
        ],
        out_specs=[
            pl.BlockSpec((rb, D), lambda i: (i, 0)),
            pl.BlockSpec((rb, D), lambda i: (i, 0)),
        ],
        out_shape=[jax.ShapeDtypeStruct((V, D), jnp.float32)] * 2,
    )(feat_table, A, W1a)

    C = pl.pallas_call(
        _ctab_kernel,
        out_shape=jax.ShapeDtypeStruct((8, D), jnp.float32),
    )(r_pad, Bm, b_gv.reshape(1, D))

    mesh = plsc.VectorSubcoreMesh(core_axis_name="c", subcore_axis_name="s")
    sc_gather = functools.partial(
        pl.kernel,
        mesh=mesh,
        out_type=(
            jax.ShapeDtypeStruct((BLS, D), jnp.float32),
            jax.ShapeDtypeStruct((BS, D), jnp.float32),
        ),
        scratch_types=[
            pltpu.VMEM((bpw, 128), jnp.int32),
            pltpu.VMEM((_CH, D), jnp.float32),
            pltpu.VMEM((_CH, D), jnp.float32),
            pltpu.VMEM((128, D), jnp.float32),
            pltpu.VMEM((max(BS // _NW // 128, 1), 128), jnp.int32),

            pltpu.SemaphoreType.DMA,
        ],
    )(_sc_gather_body)

    rbl = 128 * L
    b1r = b1.reshape(1, D)
    outs = []
    for s in range(_NSLICE):
        G, S1 = sc_gather(P, F1, uv_i[s], nodes_i[s])
        outs.append(pl.pallas_call(
            _combine_kernel,
            grid=(BS // 128,),
            in_specs=[
                pl.BlockSpec((rbl, D), lambda i: (i, 0)),
                pl.BlockSpec((1, 1, rbl), lambda i: (i, 0, 0)),
                pl.BlockSpec((8, D), lambda i: (0, 0)),
                pl.BlockSpec((128, D), lambda i: (i, 0)),
                pl.BlockSpec((D, D), lambda i: (0, 0)),
                pl.BlockSpec((1, D), lambda i: (0, 0)),
            ],
            out_specs=pl.BlockSpec((128, D), lambda i: (i, 0)),
            out_shape=jax.ShapeDtypeStruct((BS, D), jnp.float32),
        )(G, r3[s], C, S1, W1b, b1r))
    return jnp.concatenate(outs, axis=0)


# speedup vs baseline: 1.1767x; 1.0010x over previous
import functools

import jax
import jax.numpy as jnp
from jax import lax
from jax.experimental import pallas as pl
from jax.experimental.pallas import tpu as pltpu
from jax.experimental.pallas import tpu_sc as plsc

D = 128
L = 32

_NC = 2
_NS = 16
_NW = _NC * _NS

_CH = 256


def _proj_kernel(feat_ref, a_ref, w1a_ref, p_ref, f1_ref):
    f = feat_ref[...]
    p_ref[...] = jnp.dot(f, a_ref[...], preferred_element_type=jnp.float32)
    f1_ref[...] = jnp.dot(f, w1a_ref[...], preferred_element_type=jnp.float32)


def _ctab_kernel(r_ref, bm_ref, bgv_ref, c_ref):
    c_ref[...] = (
        jnp.dot(r_ref[...], bm_ref[...], preferred_element_type=jnp.float32)
        + bgv_ref[...]
    )


def _combine_kernel(g_ref, r_ref, c_ref, s1_ref, w1b_ref, b1_ref, out_ref):
    g = g_ref[...]
    r = r_ref[0, 0, :]
    rc = jnp.take_along_axis(
        c_ref[...], jnp.broadcast_to(r[:, None], (r.shape[0], D)), axis=0)
    h = jnp.maximum(g + rc, 0.0)
    neigh = jnp.sum(h.reshape(-1, L, D), axis=1) * (1.0 / L)
    comb = (s1_ref[...]
            + jnp.dot(neigh, w1b_ref[...], preferred_element_type=jnp.float32)
            + b1_ref[...])
    out_ref[...] = jnp.maximum(comb, 0.0)


def _sc_gather_body(p_hbm, f1_hbm, uv_hbm, nodes_hbm, g_out, s_out,
                    uvidx_v, buf_a, buf_b, rows_s, nidx_v, sem):
    wid = lax.axis_index("s") * _NC + lax.axis_index("c")
    bpw = uv_hbm.shape[1]
    nch = bpw * 128 // _CH
    spw = nodes_hbm.shape[0] * 128 // _NW
    ipc = _CH // 128

    pltpu.sync_copy(uv_hbm.at[wid], uvidx_v)

    def fire(buf, t):
        for j in range(ipc):
            pltpu.async_copy(p_hbm.at[uvidx_v.at[t * ipc + j]],
                             buf.at[pl.ds(j * 128, 128)], sem)

    def drain(buf):
        pltpu.make_async_copy(p_hbm.at[pl.ds(0, _CH)], buf, sem).wait()

    def writeback(buf, t):
        pltpu.sync_copy(buf, g_out.at[pl.ds(wid * bpw * 128 + t * _CH, _CH)])

    fire(buf_a, 0)

    def two_chunks(i, carry):
        t0 = i * 2
        fire(buf_b, t0 + 1)
        drain(buf_a)
        writeback(buf_a, t0)

        @pl.when(i < nch // 2 - 1)
        def _():
            fire(buf_a, t0 + 2)

        drain(buf_b)
        writeback(buf_b, t0 + 1)
        return carry

    lax.fori_loop(0, nch // 2, two_chunks, 0)

    pltpu.sync_copy(nodes_hbm.at[pl.ds(wid * (spw // 128), spw // 128)],
                    nidx_v)
    for h in range(spw // 128):
        pltpu.async_copy(f1_hbm.at[nidx_v.at[h]], rows_s, sem)
        pltpu.make_async_copy(f1_hbm.at[pl.ds(0, 128)], rows_s, sem).wait()
        pltpu.sync_copy(rows_s, s_out.at[pl.ds(wid * spw + h * 128, 128)])


_NSLICE = 4


def kernel(nodes, history_uv, history_r, feat_table, r_table, W_gv, b_gv, W1, b1):
    B = nodes.shape[0]
    V = feat_table.shape[0]
    BS = B // _NSLICE
    BLS = BS * L
    bpw = BLS // _NW // 128

    nodes_i = nodes.astype(jnp.int32).reshape(_NSLICE, BS // 128, 128)
    uv_i = history_uv.astype(jnp.int32).reshape(_NSLICE, _NW, bpw, 128)
    r3 = history_r.astype(jnp.int32).reshape(_NSLICE, BS // 128, 1, 128 * L)

    A = W_gv[:D]
    Bm = W_gv[D:]
    W1a = W1[:D]
    W1b = W1[D:]
    r_pad = jnp.pad(r_table, ((0, 8 - r_table.shape[0]), (0, 0)))

    rb = 10000
    P, F1 = pl.pallas_call(
        _proj_kernel,
        grid=(V // rb,),
        in_specs=[
            pl.BlockSpec((rb, D), lambda i: (i, 0)),
            pl.BlockSpec((D, D), lambda i: (0, 0)),
            pl.BlockSpec((D, D), lambda i: (0, 0)),
        ],
        out_specs=[
            pl.BlockSpec((rb, D), lambda i: (i, 0)),
            pl.BlockSpec((rb, D), lambda i: (i, 0)),
        ],
        out_shape=[jax.ShapeDtypeStruct((V, D), jnp.float32)] * 2,
    )(feat_table, A, W1a)

    C = pl.pallas_call(
        _ctab_kernel,
        out_shape=jax.ShapeDtypeStruct((8, D), jnp.float32),
    )(r_pad, Bm, b_gv.reshape(1, D))

    mesh = plsc.VectorSubcoreMesh(core_axis_name="c", subcore_axis_name="s")
    sc_gather = functools.partial(
        pl.kernel,
        mesh=mesh,
        out_type=(
            jax.ShapeDtypeStruct((BLS, D), jnp.float32),
            jax.ShapeDtypeStruct((BS, D), jnp.float32),
        ),
        scratch_types=[
            pltpu.VMEM((bpw, 128), jnp.int32),
            pltpu.VMEM((_CH, D), jnp.float32),
            pltpu.VMEM((_CH, D), jnp.float32),
            pltpu.VMEM((128, D), jnp.float32),
            pltpu.VMEM((max(BS // _NW // 128, 1), 128), jnp.int32),

            pltpu.SemaphoreType.DMA,
        ],
    )(_sc_gather_body)

    rbl = 128 * L
    b1r = b1.reshape(1, D)
    outs = []
    for s in range(_NSLICE):
        G, S1 = sc_gather(P, F1, uv_i[s], nodes_i[s])
        outs.append(pl.pallas_call(
            _combine_kernel,
            grid=(BS // 128,),
            in_specs=[
                pl.BlockSpec((rbl, D), lambda i: (i, 0)),
                pl.BlockSpec((1, 1, rbl), lambda i: (i, 0, 0)),
                pl.BlockSpec((8, D), lambda i: (0, 0)),
                pl.BlockSpec((128, D), lambda i: (i, 0)),
                pl.BlockSpec((D, D), lambda i: (0, 0)),
                pl.BlockSpec((1, D), lambda i: (0, 0)),
            ],
            out_specs=pl.BlockSpec((128, D), lambda i: (i, 0)),
            out_shape=jax.ShapeDtypeStruct((BS, D), jnp.float32),
        )(G, r3[s], C, S1, W1b, b1r))
    return jnp.concatenate(outs, axis=0)
